# baseline (speedup 1.0000x reference)
import functools

import jax
import jax.numpy as jnp
from jax import lax
from jax.experimental import pallas as pl
from jax.experimental.pallas import tpu as pltpu
from jax.experimental.pallas import tpu_sc as plsc

NC = 2
NS = 16
HALF = 32


def _make_spmm(n_pad, chunks_per_tile):
    mesh = plsc.VectorSubcoreMesh(
        core_axis_name="c", subcore_axis_name="s", num_cores=NC, num_subcores=NS)
    rows_per_tile = n_pad // NS
    U = chunks_per_tile // 8

    @functools.partial(
        pl.kernel,
        out_type=jax.ShapeDtypeStruct((NC * n_pad, HALF), jnp.float32),
        mesh=mesh,
        compiler_params=pltpu.CompilerParams(use_tc_tiling_on_sc=False,
                                             needs_layout_passes=False),
        scratch_types=[
            pltpu.VMEM_SHARED((n_pad, HALF), jnp.float32),
            pltpu.VMEM((4, 3, 128), jnp.int32),
            pltpu.VMEM((4, 128, HALF), jnp.float32),
            [pltpu.SemaphoreType.DMA] * 4,
            [pltpu.SemaphoreType.DMA] * 4,
            [pltpu.SemaphoreType.DMA] * 4,
        ],
    )
    def spmm(xt0, xt1, epack, zeros, out, acc, ebuf, gbuf, sem_i, sem_g, sem_s):
        c = lax.axis_index("c")
        t = lax.axis_index("s")
        rbase = t * rows_per_tile
        pltpu.sync_copy(zeros.at[pl.ds(rbase, rows_per_tile)],
                        acc.at[pl.ds(rbase, rows_per_tile)])
        plsc.subcore_barrier()

        ebase = t * chunks_per_tile
        lane_consts = [jnp.full((16,), e2, jnp.int32) for e2 in range(16)]

        def fire_idx(g, p):
            pltpu.async_copy(epack.at[ebase + g], ebuf.at[p], sem_i[p])

        def wait_idx(p):
            pltpu.make_async_copy(epack.at[ebase], ebuf.at[p],
                                  sem_i[p]).wait()

        def fire_gather(p):
            @pl.when(c == 0)
            def _():
                pltpu.async_copy(xt0.at[ebuf.at[p, 0]], gbuf.at[p], sem_g[p])

            @pl.when(c == 1)
            def _():
                pltpu.async_copy(xt1.at[ebuf.at[p, 0]], gbuf.at[p], sem_g[p])

        def wait_gather(p):
            pltpu.make_async_copy(xt0.at[ebuf.at[p, 0]], gbuf.at[p],
                                  sem_g[p]).wait()

        def fire_scatter(p):
            pltpu.async_copy(gbuf.at[p], acc.at[ebuf.at[p, 1]], sem_s[p],
                             add=True)

        def wait_scatter(p):
            pltpu.make_async_copy(gbuf.at[p], acc.at[ebuf.at[p, 1]],
                                  sem_s[p]).wait()

        def scale(p):
            for q in range(8):
                vi = ebuf[p, 2, pl.ds(q * 16, 16)]
                vals16 = plsc.bitcast(vi, jnp.float32)
                for e2 in range(16):
                    splat = jnp.take_along_axis(vals16, lane_consts[e2],
                                                axis=0)
                    r = q * 16 + e2
                    gbuf[p, r, pl.ds(0, 16)] = (
                        gbuf[p, r, pl.ds(0, 16)] * splat)
                    gbuf[p, r, pl.ds(16, 16)] = (
                        gbuf[p, r, pl.ds(16, 16)] * splat)

        for g0 in range(2):
            fire_idx(g0, g0)
        for g0 in range(2):
            wait_idx(g0)
            fire_gather(g0)

        n_ch = chunks_per_tile

        def ubody(u, carry):
            for j in range(4):
                g = 4 * u + j
                p = j
                p2 = (j + 2) % 4

                @pl.when(g >= 2)
                def _():
                    wait_scatter(p2)

                @pl.when(g + 2 < n_ch)
                def _():
                    fire_idx(g + 2, p2)

                wait_gather(p)
                scale(p)

                @pl.when(g + 2 < n_ch)
                def _():
                    wait_idx(p2)
                    fire_gather(p2)

                fire_scatter(p)
            return carry

        lax.fori_loop(0, n_ch // 4, ubody, 0)
        wait_scatter(2)
        wait_scatter(3)

        plsc.subcore_barrier()
        pltpu.sync_copy(acc.at[pl.ds(rbase, rows_per_tile)],
                        out.at[pl.ds(c * n_pad + rbase, rows_per_tile)])

    return spmm


def _dense_layer(sp, x0p, x1p, wstack, bdone, bias, n_pad):
    blk = 256
    nblk = n_pad // 4 // blk

    def body(sa_ref, sb_ref, xa_ref, xb_ref, w_ref, n_ref, b_ref,
             o0_ref, o1_ref):
        sa = sa_ref[...]
        sb = sb_ref[...]
        ia = sa * xa_ref[...]
        ib = sb * xb_ref[...]

        def mm(a, w):
            return lax.dot_general(a, w, (((1,), (0,)), ((), ())),
                                   preferred_element_type=jnp.float32)

        y0 = (mm(sa, w_ref[0]) + mm(sb, w_ref[1]) + mm(ia, w_ref[2])
              + mm(ib, w_ref[3]) + b_ref[0])
        y1 = (mm(sa, w_ref[4]) + mm(sb, w_ref[5]) + mm(ia, w_ref[6])
              + mm(ib, w_ref[7]) + b_ref[1])
        y0 = jnp.where(y0 > 0, y0, 0.2 * y0)
        y1 = jnp.where(y1 > 0, y1, 0.2 * y1)
        n2 = mm(y0 * y0 + y1 * y1, n_ref[...])
        inv = lax.rsqrt(jnp.maximum(n2, 1e-24))
        o0_ref[...] = y0 * inv
        o1_ref[...] = y1 * inv

    return pl.pallas_call(
        body,
        grid=(nblk,),
        in_specs=[
            pl.BlockSpec((blk, 128), lambda i: (i, 0)),
            pl.BlockSpec((blk, 128), lambda i: (i + nblk, 0)),
            pl.BlockSpec((blk, 128), lambda i: (i, 0)),
            pl.BlockSpec((blk, 128), lambda i: (i, 0)),
            pl.BlockSpec((8, 128, 128), lambda i: (0, 0, 0)),
            pl.BlockSpec((128, 128), lambda i: (0, 0)),
            pl.BlockSpec((2, 1, 128), lambda i: (0, 0, 0)),
        ],
        out_specs=(pl.BlockSpec((blk, 128), lambda i: (i, 0)),
                   pl.BlockSpec((blk, 128), lambda i: (i, 0))),
        out_shape=(jax.ShapeDtypeStruct((n_pad // 4, 128), jnp.float32),
                   jax.ShapeDtypeStruct((n_pad // 4, 128), jnp.float32)),
    )(sp, sp, x0p, x1p, wstack, bdone, bias)


def _make_gather(n_pad, bsz):
    mesh = plsc.VectorSubcoreMesh(
        core_axis_name="c", subcore_axis_name="s", num_cores=NC, num_subcores=NS)
    per_w = bsz // (NC * NS)

    @functools.partial(
        pl.kernel,
        out_type=jax.ShapeDtypeStruct((4, 3, bsz, 64), jnp.float32),
        mesh=mesh,
        compiler_params=pltpu.CompilerParams(use_tc_tiling_on_sc=False),
        scratch_types=[
            pltpu.VMEM((128,), jnp.int32),
            pltpu.VMEM((128, HALF), jnp.float32),
            pltpu.SemaphoreType.DMA,
        ],
    )
    def gk(t00, t01, t10, t11, t20, t21, t30, t31, idx, out, idx_v, gb, sem):
        c = lax.axis_index("c")
        s = lax.axis_index("s")
        w = s * NC + c
        tabs = [[t00, t01], [t10, t11], [t20, t21], [t30, t31]]
        for k in range(4):
            for r in range(3):
                pltpu.sync_copy(idx.at[r, w], idx_v)
                for h in range(2):
                    pltpu.async_copy(tabs[k][h].at[idx_v], gb, sem).wait()
                    pltpu.sync_copy(
                        gb, out.at[k, r, pl.ds(w * 128, 128),
                                   pl.ds(h * HALF, HALF)])

    return gk


def _loss_tc(g2, bsz, bdone2):
    def body(g_ref, n_ref, loss_ref, bpr_ref):
        m = jnp.zeros((bsz // 2, 128), jnp.float32)
        for k in range(4):
            u = g_ref[k, 0]
            m = m + u * (g_ref[k, 1] - g_ref[k, 2])
        z = lax.dot_general(m, n_ref[...], (((1,), (0,)), ((), ())),
                            preferred_element_type=jnp.float32)
        sp = jnp.maximum(-z, 0.0) + jnp.log(1.0 + jnp.exp(-jnp.abs(z)))
        bpr = jnp.sum(sp) / float(64 * bsz)
        reg = (jnp.sum(g_ref[0, 0] * g_ref[0, 0])
               + jnp.sum(g_ref[0, 1] * g_ref[0, 1])
               + jnp.sum(g_ref[0, 2] * g_ref[0, 2])) / float(bsz)
        loss_ref[...] = jnp.reshape(bpr + 1e-4 * reg, (1, 1))
        bpr_ref[...] = jnp.reshape(bpr, (1, 1))

    return pl.pallas_call(
        body,
        out_shape=(jax.ShapeDtypeStruct((1, 1), jnp.float32),
                   jax.ShapeDtypeStruct((1, 1), jnp.float32)),
    )(g2, bdone2)


def kernel(user_emb, item_emb, W1, b1, W2, b2, adj_values, adj_indices,
           users, pos_items, neg_items):
    n_users, d = user_emb.shape
    n_items = item_emb.shape[0]
    n = n_users + n_items
    n_pad = ((n + 1023) // 1024) * 1024
    e = adj_values.shape[0]
    e_pad = ((e + 16383) // 16384) * 16384
    crows = e_pad // 128
    chunks_per_tile = crows // NS
    bsz = users.shape[0]

    row = adj_indices[0].astype(jnp.int32)
    col = adj_indices[1].astype(jnp.int32)
    npad_rows = n_pad - n
    ar = jnp.arange(e_pad - e, dtype=jnp.int32)
    spread = n + (ar % npad_rows)
    row_p = jnp.concatenate([row, spread])
    col_p = jnp.concatenate([col, spread])
    val_p = jnp.concatenate([adj_values, jnp.zeros((e_pad - e,), jnp.float32)])
    epack = jnp.stack(
        [col_p.reshape(crows, 128), row_p.reshape(crows, 128),
         lax.bitcast_convert_type(val_p, jnp.int32).reshape(crows, 128)],
        axis=1)
    zeros = jnp.zeros((n_pad, HALF), jnp.float32)

    all_emb = jnp.concatenate([user_emb, item_emb], axis=0)
    all_pad = jnp.pad(all_emb, ((0, npad_rows), (0, 0)))
    x0p = all_pad[:, :HALF].reshape(n_pad // 4, 128)
    x1p = all_pad[:, HALF:].reshape(n_pad // 4, 128)

    eye4 = jnp.eye(4, dtype=jnp.float32)
    bdone = jnp.kron(eye4, jnp.ones((HALF, HALF), jnp.float32))
    bdone2 = jnp.kron(jnp.eye(2, dtype=jnp.float32),
                      jnp.ones((64, 64), jnp.float32))

    spmm = _make_spmm(n_pad, chunks_per_tile)
    halves = [(x0p, x1p)]
    for k in range(W1.shape[0]):
        xa, xb = halves[-1]
        sp = spmm(xa.reshape(n_pad, HALF), xb.reshape(n_pad, HALF),
                  epack, zeros)
        blocks = []
        for c in range(2):
            for w in (W1[k], W2[k]):
                blocks.append(jnp.kron(
                    eye4, w[HALF * c:HALF * (c + 1), :HALF].T))
                blocks.append(jnp.kron(
                    eye4, w[HALF * c:HALF * (c + 1), HALF:].T))
        wstack = jnp.stack(blocks)
        bsum = b1[k] + b2[k]
        bias = jnp.stack([jnp.tile(bsum[:HALF], 4), jnp.tile(bsum[HALF:], 4)])
        bias = bias.reshape(2, 1, 128)
        o0, o1 = _dense_layer(sp.reshape(2 * n_pad // 4, 128), xa, xb,
                              wstack, bdone, bias, n_pad)
        halves.append((o0, o1))

    idx_u = users.astype(jnp.int32)
    idx_p = pos_items.astype(jnp.int32) + n_users
    idx_n = neg_items.astype(jnp.int32) + n_users
    idx = jnp.stack([idx_u, idx_p, idx_n]).reshape(3, bsz // 128, 128)

    tables = []
    for (a, b) in halves:
        tables.append(a.reshape(n_pad, HALF))
        tables.append(b.reshape(n_pad, HALF))
    g = _make_gather(n_pad, bsz)(*tables, idx)
    loss_a, bpr_a = _loss_tc(g.reshape(4, 3, bsz // 2, 128), bsz, bdone2)
    return (loss_a[0, 0], bpr_a[0, 0])


# speedup vs baseline: 1.2987x; 1.2987x over previous
import functools

import jax
import jax.numpy as jnp
from jax import lax
from jax.experimental import pallas as pl
from jax.experimental.pallas import tpu as pltpu
from jax.experimental.pallas import tpu_sc as plsc

NC = 2
NS = 16
HALF = 32


def _make_spmm(n_pad, chunks_per_tile):
    mesh = plsc.VectorSubcoreMesh(
        core_axis_name="c", subcore_axis_name="s", num_cores=NC, num_subcores=NS)
    rows_per_tile = n_pad // NS
    U = chunks_per_tile // 8

    @functools.partial(
        pl.kernel,
        out_type=jax.ShapeDtypeStruct((NC * n_pad, HALF), jnp.float32),
        mesh=mesh,
        compiler_params=pltpu.CompilerParams(use_tc_tiling_on_sc=False,
                                             needs_layout_passes=False),
        scratch_types=[
            pltpu.VMEM_SHARED((n_pad, HALF), jnp.float32),
            pltpu.VMEM((8, 3, 128), jnp.int32),
            pltpu.VMEM((4, 128, HALF), jnp.float32),
            [pltpu.SemaphoreType.DMA] * 8,
            [pltpu.SemaphoreType.DMA] * 4,
            [pltpu.SemaphoreType.DMA] * 4,
        ],
    )
    def spmm(xt0, xt1, epack, zeros, out, acc, ebuf, gbuf, sem_i, sem_g, sem_s):
        c = lax.axis_index("c")
        t = lax.axis_index("s")
        rbase = t * rows_per_tile
        pltpu.sync_copy(zeros.at[pl.ds(rbase, rows_per_tile)],
                        acc.at[pl.ds(rbase, rows_per_tile)])
        plsc.subcore_barrier()

        ebase = t * chunks_per_tile
        lane_consts = [jnp.full((16,), e2, jnp.int32) for e2 in range(16)]

        def fire_idx(g, p8):
            pltpu.async_copy(epack.at[ebase + g], ebuf.at[p8], sem_i[p8])

        def wait_idx(p8):
            pltpu.make_async_copy(epack.at[ebase], ebuf.at[p8],
                                  sem_i[p8]).wait()

        def fire_gather(p8, p4):
            @pl.when(c == 0)
            def _():
                pltpu.async_copy(xt0.at[ebuf.at[p8, 0]], gbuf.at[p4],
                                 sem_g[p4])

            @pl.when(c == 1)
            def _():
                pltpu.async_copy(xt1.at[ebuf.at[p8, 0]], gbuf.at[p4],
                                 sem_g[p4])

        def wait_gather(p8, p4):
            pltpu.make_async_copy(xt0.at[ebuf.at[p8, 0]], gbuf.at[p4],
                                  sem_g[p4]).wait()

        def fire_scatter(p8, p4):
            pltpu.async_copy(gbuf.at[p4], acc.at[ebuf.at[p8, 1]], sem_s[p4],
                             add=True)

        def wait_scatter(p8, p4):
            pltpu.make_async_copy(gbuf.at[p4], acc.at[ebuf.at[p8, 1]],
                                  sem_s[p4]).wait()

        def scale(p8, p4):
            def sbody(q, _):
                vi = ebuf[p8, 2, pl.ds(q * 16, 16)]
                vals16 = plsc.bitcast(vi, jnp.float32)
                rb = q * 16
                for e2 in range(16):
                    splat = jnp.take_along_axis(vals16, lane_consts[e2],
                                                axis=0)
                    r = rb + e2
                    gbuf[p4, r, pl.ds(0, 16)] = (
                        gbuf[p4, r, pl.ds(0, 16)] * splat)
                    gbuf[p4, r, pl.ds(16, 16)] = (
                        gbuf[p4, r, pl.ds(16, 16)] * splat)
                return 0
            lax.fori_loop(0, 8, sbody, 0)

        for g0 in range(4):
            fire_idx(g0, g0)
        for g0 in range(2):
            wait_idx(g0)
            fire_gather(g0, g0)

        def ubody(u, carry):
            for j in range(8):
                g = 8 * u + j
                p4 = j % 4
                p8 = j
                pa8 = (j + 4) % 8
                pb8 = (j + 2) % 8
                pb4 = (j + 2) % 4

                @pl.when(g + 4 < chunks_per_tile)
                def _():
                    fire_idx(g + 4, pa8)

                @pl.when(jnp.logical_and(g + 2 < chunks_per_tile, g >= 2))
                def _():
                    wait_scatter(pb8, pb4)

                @pl.when(g + 2 < chunks_per_tile)
                def _():
                    wait_idx(pb8)
                    fire_gather(pb8, pb4)

                wait_gather(p8, p4)
                scale(p8, p4)
                fire_scatter(p8, p4)
            return carry

        lax.fori_loop(0, U, ubody, 0)
        for j in range(4):
            wait_scatter((j + 4) % 8, j)

        plsc.subcore_barrier()
        pltpu.sync_copy(acc.at[pl.ds(rbase, rows_per_tile)],
                        out.at[pl.ds(c * n_pad + rbase, rows_per_tile)])

    return spmm


def _dense_layer(sp, x0p, x1p, wstack, bdone, bias, n_pad):
    blk = 256
    nblk = n_pad // 4 // blk

    def body(sa_ref, sb_ref, xa_ref, xb_ref, w_ref, n_ref, b_ref,
             o0_ref, o1_ref):
        sa = sa_ref[...]
        sb = sb_ref[...]
        ia = sa * xa_ref[...]
        ib = sb * xb_ref[...]

        def mm(a, w):
            return lax.dot_general(a, w, (((1,), (0,)), ((), ())),
                                   preferred_element_type=jnp.float32)

        y0 = (mm(sa, w_ref[0]) + mm(sb, w_ref[1]) + mm(ia, w_ref[2])
              + mm(ib, w_ref[3]) + b_ref[0])
        y1 = (mm(sa, w_ref[4]) + mm(sb, w_ref[5]) + mm(ia, w_ref[6])
              + mm(ib, w_ref[7]) + b_ref[1])
        y0 = jnp.where(y0 > 0, y0, 0.2 * y0)
        y1 = jnp.where(y1 > 0, y1, 0.2 * y1)
        n2 = mm(y0 * y0 + y1 * y1, n_ref[...])
        inv = lax.rsqrt(jnp.maximum(n2, 1e-24))
        o0_ref[...] = y0 * inv
        o1_ref[...] = y1 * inv

    return pl.pallas_call(
        body,
        grid=(nblk,),
        in_specs=[
            pl.BlockSpec((blk, 128), lambda i: (i, 0)),
            pl.BlockSpec((blk, 128), lambda i: (i + nblk, 0)),
            pl.BlockSpec((blk, 128), lambda i: (i, 0)),
            pl.BlockSpec((blk, 128), lambda i: (i, 0)),
            pl.BlockSpec((8, 128, 128), lambda i: (0, 0, 0)),
            pl.BlockSpec((128, 128), lambda i: (0, 0)),
            pl.BlockSpec((2, 1, 128), lambda i: (0, 0, 0)),
        ],
        out_specs=(pl.BlockSpec((blk, 128), lambda i: (i, 0)),
                   pl.BlockSpec((blk, 128), lambda i: (i, 0))),
        out_shape=(jax.ShapeDtypeStruct((n_pad // 4, 128), jnp.float32),
                   jax.ShapeDtypeStruct((n_pad // 4, 128), jnp.float32)),
    )(sp, sp, x0p, x1p, wstack, bdone, bias)


def _make_gather(n_pad, bsz):
    mesh = plsc.VectorSubcoreMesh(
        core_axis_name="c", subcore_axis_name="s", num_cores=NC, num_subcores=NS)
    per_w = bsz // (NC * NS)

    @functools.partial(
        pl.kernel,
        out_type=jax.ShapeDtypeStruct((4, 3, bsz, 64), jnp.float32),
        mesh=mesh,
        compiler_params=pltpu.CompilerParams(use_tc_tiling_on_sc=False),
        scratch_types=[
            pltpu.VMEM((128,), jnp.int32),
            pltpu.VMEM((128, HALF), jnp.float32),
            pltpu.SemaphoreType.DMA,
        ],
    )
    def gk(t00, t01, t10, t11, t20, t21, t30, t31, idx, out, idx_v, gb, sem):
        c = lax.axis_index("c")
        s = lax.axis_index("s")
        w = s * NC + c
        tabs = [[t00, t01], [t10, t11], [t20, t21], [t30, t31]]
        for k in range(4):
            for r in range(3):
                pltpu.sync_copy(idx.at[r, w], idx_v)
                for h in range(2):
                    pltpu.async_copy(tabs[k][h].at[idx_v], gb, sem).wait()
                    pltpu.sync_copy(
                        gb, out.at[k, r, pl.ds(w * 128, 128),
                                   pl.ds(h * HALF, HALF)])

    return gk


def _loss_tc(g2, bsz, bdone2):
    def body(g_ref, n_ref, loss_ref, bpr_ref):
        m = jnp.zeros((bsz // 2, 128), jnp.float32)
        for k in range(4):
            u = g_ref[k, 0]
            m = m + u * (g_ref[k, 1] - g_ref[k, 2])
        z = lax.dot_general(m, n_ref[...], (((1,), (0,)), ((), ())),
                            preferred_element_type=jnp.float32)
        sp = jnp.maximum(-z, 0.0) + jnp.log(1.0 + jnp.exp(-jnp.abs(z)))
        bpr = jnp.sum(sp) / float(64 * bsz)
        reg = (jnp.sum(g_ref[0, 0] * g_ref[0, 0])
               + jnp.sum(g_ref[0, 1] * g_ref[0, 1])
               + jnp.sum(g_ref[0, 2] * g_ref[0, 2])) / float(bsz)
        loss_ref[...] = jnp.reshape(bpr + 1e-4 * reg, (1, 1))
        bpr_ref[...] = jnp.reshape(bpr, (1, 1))

    return pl.pallas_call(
        body,
        out_shape=(jax.ShapeDtypeStruct((1, 1), jnp.float32),
                   jax.ShapeDtypeStruct((1, 1), jnp.float32)),
    )(g2, bdone2)


def kernel(user_emb, item_emb, W1, b1, W2, b2, adj_values, adj_indices,
           users, pos_items, neg_items):
    n_users, d = user_emb.shape
    n_items = item_emb.shape[0]
    n = n_users + n_items
    n_pad = ((n + 1023) // 1024) * 1024
    e = adj_values.shape[0]
    e_pad = ((e + 16383) // 16384) * 16384
    crows = e_pad // 128
    chunks_per_tile = crows // NS
    bsz = users.shape[0]

    row = adj_indices[0].astype(jnp.int32)
    col = adj_indices[1].astype(jnp.int32)
    npad_rows = n_pad - n
    ar = jnp.arange(e_pad - e, dtype=jnp.int32)
    spread = n + (ar % npad_rows)
    row_p = jnp.concatenate([row, spread])
    col_p = jnp.concatenate([col, spread])
    val_p = jnp.concatenate([adj_values, jnp.zeros((e_pad - e,), jnp.float32)])
    epack = jnp.stack(
        [col_p.reshape(crows, 128), row_p.reshape(crows, 128),
         lax.bitcast_convert_type(val_p, jnp.int32).reshape(crows, 128)],
        axis=1)
    zeros = jnp.zeros((n_pad, HALF), jnp.float32)

    all_emb = jnp.concatenate([user_emb, item_emb], axis=0)
    all_pad = jnp.pad(all_emb, ((0, npad_rows), (0, 0)))
    x0p = all_pad[:, :HALF].reshape(n_pad // 4, 128)
    x1p = all_pad[:, HALF:].reshape(n_pad // 4, 128)

    eye4 = jnp.eye(4, dtype=jnp.float32)
    bdone = jnp.kron(eye4, jnp.ones((HALF, HALF), jnp.float32))
    bdone2 = jnp.kron(jnp.eye(2, dtype=jnp.float32),
                      jnp.ones((64, 64), jnp.float32))

    spmm = _make_spmm(n_pad, chunks_per_tile)
    halves = [(x0p, x1p)]
    for k in range(W1.shape[0]):
        xa, xb = halves[-1]
        sp = spmm(xa.reshape(n_pad, HALF), xb.reshape(n_pad, HALF),
                  epack, zeros)
        blocks = []
        for c in range(2):
            for w in (W1[k], W2[k]):
                blocks.append(jnp.kron(
                    eye4, w[HALF * c:HALF * (c + 1), :HALF].T))
                blocks.append(jnp.kron(
                    eye4, w[HALF * c:HALF * (c + 1), HALF:].T))
        wstack = jnp.stack(blocks)
        bsum = b1[k] + b2[k]
        bias = jnp.stack([jnp.tile(bsum[:HALF], 4), jnp.tile(bsum[HALF:], 4)])
        bias = bias.reshape(2, 1, 128)
        o0, o1 = _dense_layer(sp.reshape(2 * n_pad // 4, 128), xa, xb,
                              wstack, bdone, bias, n_pad)
        halves.append((o0, o1))

    idx_u = users.astype(jnp.int32)
    idx_p = pos_items.astype(jnp.int32) + n_users
    idx_n = neg_items.astype(jnp.int32) + n_users
    idx = jnp.stack([idx_u, idx_p, idx_n]).reshape(3, bsz // 128, 128)

    tables = []
    for (a, b) in halves:
        tables.append(a.reshape(n_pad, HALF))
        tables.append(b.reshape(n_pad, HALF))
    g = _make_gather(n_pad, bsz)(*tables, idx)
    loss_a, bpr_a = _loss_tc(g.reshape(4, 3, bsz // 2, 128), bsz, bdone2)
    return (loss_a[0, 0], bpr_a[0, 0])


# speedup vs baseline: 1.3158x; 1.0131x over previous
import functools

import jax
import jax.numpy as jnp
from jax import lax
from jax.experimental import pallas as pl
from jax.experimental.pallas import tpu as pltpu
from jax.experimental.pallas import tpu_sc as plsc

NC = 2
NS = 16
HALF = 32


def _make_spmm(n_pad, chunks_per_tile):
    mesh = plsc.VectorSubcoreMesh(
        core_axis_name="c", subcore_axis_name="s", num_cores=NC, num_subcores=NS)
    rows_per_tile = n_pad // NS
    U = chunks_per_tile // 8

    @functools.partial(
        pl.kernel,
        out_type=jax.ShapeDtypeStruct((NC * n_pad, HALF), jnp.float32),
        mesh=mesh,
        compiler_params=pltpu.CompilerParams(use_tc_tiling_on_sc=False,
                                             needs_layout_passes=False),
        scratch_types=[
            pltpu.VMEM_SHARED((n_pad, HALF), jnp.float32),
            pltpu.VMEM((8, 3, 128), jnp.int32),
            pltpu.VMEM((4, 128, HALF), jnp.float32),
            [pltpu.SemaphoreType.DMA] * 8,
            [pltpu.SemaphoreType.DMA] * 4,
            [pltpu.SemaphoreType.DMA] * 4,
        ],
    )
    def spmm(xt0, xt1, epack, zeros, out, acc, ebuf, gbuf, sem_i, sem_g, sem_s):
        c = lax.axis_index("c")
        t = lax.axis_index("s")
        rbase = t * rows_per_tile
        pltpu.sync_copy(zeros.at[pl.ds(rbase, rows_per_tile)],
                        acc.at[pl.ds(rbase, rows_per_tile)])
        plsc.subcore_barrier()

        ebase = t * chunks_per_tile
        lane_consts = [jnp.full((16,), e2, jnp.int32) for e2 in range(16)]

        def fire_idx(g, p8):
            pltpu.async_copy(epack.at[ebase + g], ebuf.at[p8], sem_i[p8])

        def wait_idx(p8):
            pltpu.make_async_copy(epack.at[ebase], ebuf.at[p8],
                                  sem_i[p8]).wait()

        def fire_gather(p8, p4):
            @pl.when(c == 0)
            def _():
                pltpu.async_copy(xt0.at[ebuf.at[p8, 0]], gbuf.at[p4],
                                 sem_g[p4])

            @pl.when(c == 1)
            def _():
                pltpu.async_copy(xt1.at[ebuf.at[p8, 0]], gbuf.at[p4],
                                 sem_g[p4])

        def wait_gather(p8, p4):
            pltpu.make_async_copy(xt0.at[ebuf.at[p8, 0]], gbuf.at[p4],
                                  sem_g[p4]).wait()

        def fire_scatter(p8, p4):
            pltpu.async_copy(gbuf.at[p4], acc.at[ebuf.at[p8, 1]], sem_s[p4],
                             add=True)

        def wait_scatter(p8, p4):
            pltpu.make_async_copy(gbuf.at[p4], acc.at[ebuf.at[p8, 1]],
                                  sem_s[p4]).wait()

        def scale(p8, p4):
            @plsc.parallel_loop(0, 8, unroll=2)
            def sbody(q):
                vi = ebuf[p8, 2, pl.ds(q * 16, 16)]
                vals16 = plsc.bitcast(vi, jnp.float32)
                rb = q * 16
                for e2 in range(16):
                    splat = jnp.take_along_axis(vals16, lane_consts[e2],
                                                axis=0)
                    r = rb + e2
                    gbuf[p4, r, pl.ds(0, 16)] = (
                        gbuf[p4, r, pl.ds(0, 16)] * splat)
                    gbuf[p4, r, pl.ds(16, 16)] = (
                        gbuf[p4, r, pl.ds(16, 16)] * splat)

        for g0 in range(4):
            fire_idx(g0, g0)
        for g0 in range(2):
            wait_idx(g0)
            fire_gather(g0, g0)

        def ubody(u, carry):
            for j in range(8):
                g = 8 * u + j
                p4 = j % 4
                p8 = j
                pa8 = (j + 4) % 8
                pb8 = (j + 2) % 8
                pb4 = (j + 2) % 4

                @pl.when(g + 4 < chunks_per_tile)
                def _():
                    fire_idx(g + 4, pa8)

                @pl.when(jnp.logical_and(g + 2 < chunks_per_tile, g >= 2))
                def _():
                    wait_scatter(pb8, pb4)

                @pl.when(g + 2 < chunks_per_tile)
                def _():
                    wait_idx(pb8)
                    fire_gather(pb8, pb4)

                wait_gather(p8, p4)
                scale(p8, p4)
                fire_scatter(p8, p4)
            return carry

        lax.fori_loop(0, U, ubody, 0)
        for j in range(4):
            wait_scatter((j + 4) % 8, j)

        plsc.subcore_barrier()
        pltpu.sync_copy(acc.at[pl.ds(rbase, rows_per_tile)],
                        out.at[pl.ds(c * n_pad + rbase, rows_per_tile)])

    return spmm


def _dense_layer(sp, x0p, x1p, wstack, bdone, bias, n_pad):
    blk = 256
    nblk = n_pad // 4 // blk

    def body(sa_ref, sb_ref, xa_ref, xb_ref, w_ref, n_ref, b_ref,
             o0_ref, o1_ref):
        sa = sa_ref[...]
        sb = sb_ref[...]
        ia = sa * xa_ref[...]
        ib = sb * xb_ref[...]

        def mm(a, w):
            return lax.dot_general(a, w, (((1,), (0,)), ((), ())),
                                   preferred_element_type=jnp.float32)

        y0 = (mm(sa, w_ref[0]) + mm(sb, w_ref[1]) + mm(ia, w_ref[2])
              + mm(ib, w_ref[3]) + b_ref[0])
        y1 = (mm(sa, w_ref[4]) + mm(sb, w_ref[5]) + mm(ia, w_ref[6])
              + mm(ib, w_ref[7]) + b_ref[1])
        y0 = jnp.where(y0 > 0, y0, 0.2 * y0)
        y1 = jnp.where(y1 > 0, y1, 0.2 * y1)
        n2 = mm(y0 * y0 + y1 * y1, n_ref[...])
        inv = lax.rsqrt(jnp.maximum(n2, 1e-24))
        o0_ref[...] = y0 * inv
        o1_ref[...] = y1 * inv

    return pl.pallas_call(
        body,
        grid=(nblk,),
        in_specs=[
            pl.BlockSpec((blk, 128), lambda i: (i, 0)),
            pl.BlockSpec((blk, 128), lambda i: (i + nblk, 0)),
            pl.BlockSpec((blk, 128), lambda i: (i, 0)),
            pl.BlockSpec((blk, 128), lambda i: (i, 0)),
            pl.BlockSpec((8, 128, 128), lambda i: (0, 0, 0)),
            pl.BlockSpec((128, 128), lambda i: (0, 0)),
            pl.BlockSpec((2, 1, 128), lambda i: (0, 0, 0)),
        ],
        out_specs=(pl.BlockSpec((blk, 128), lambda i: (i, 0)),
                   pl.BlockSpec((blk, 128), lambda i: (i, 0))),
        out_shape=(jax.ShapeDtypeStruct((n_pad // 4, 128), jnp.float32),
                   jax.ShapeDtypeStruct((n_pad // 4, 128), jnp.float32)),
    )(sp, sp, x0p, x1p, wstack, bdone, bias)


def _make_gather(n_pad, bsz):
    mesh = plsc.VectorSubcoreMesh(
        core_axis_name="c", subcore_axis_name="s", num_cores=NC, num_subcores=NS)
    per_w = bsz // (NC * NS)

    @functools.partial(
        pl.kernel,
        out_type=jax.ShapeDtypeStruct((4, 3, bsz, 64), jnp.float32),
        mesh=mesh,
        compiler_params=pltpu.CompilerParams(use_tc_tiling_on_sc=False),
        scratch_types=[
            pltpu.VMEM((128,), jnp.int32),
            pltpu.VMEM((128, HALF), jnp.float32),
            pltpu.SemaphoreType.DMA,
        ],
    )
    def gk(t00, t01, t10, t11, t20, t21, t30, t31, idx, out, idx_v, gb, sem):
        c = lax.axis_index("c")
        s = lax.axis_index("s")
        w = s * NC + c
        tabs = [[t00, t01], [t10, t11], [t20, t21], [t30, t31]]
        for k in range(4):
            for r in range(3):
                pltpu.sync_copy(idx.at[r, w], idx_v)
                for h in range(2):
                    pltpu.async_copy(tabs[k][h].at[idx_v], gb, sem).wait()
                    pltpu.sync_copy(
                        gb, out.at[k, r, pl.ds(w * 128, 128),
                                   pl.ds(h * HALF, HALF)])

    return gk


def _loss_tc(g2, bsz, bdone2):
    def body(g_ref, n_ref, loss_ref, bpr_ref):
        m = jnp.zeros((bsz // 2, 128), jnp.float32)
        for k in range(4):
            u = g_ref[k, 0]
            m = m + u * (g_ref[k, 1] - g_ref[k, 2])
        z = lax.dot_general(m, n_ref[...], (((1,), (0,)), ((), ())),
                            preferred_element_type=jnp.float32)
        sp = jnp.maximum(-z, 0.0) + jnp.log(1.0 + jnp.exp(-jnp.abs(z)))
        bpr = jnp.sum(sp) / float(64 * bsz)
        reg = (jnp.sum(g_ref[0, 0] * g_ref[0, 0])
               + jnp.sum(g_ref[0, 1] * g_ref[0, 1])
               + jnp.sum(g_ref[0, 2] * g_ref[0, 2])) / float(bsz)
        loss_ref[...] = jnp.reshape(bpr + 1e-4 * reg, (1, 1))
        bpr_ref[...] = jnp.reshape(bpr, (1, 1))

    return pl.pallas_call(
        body,
        out_shape=(jax.ShapeDtypeStruct((1, 1), jnp.float32),
                   jax.ShapeDtypeStruct((1, 1), jnp.float32)),
    )(g2, bdone2)


def kernel(user_emb, item_emb, W1, b1, W2, b2, adj_values, adj_indices,
           users, pos_items, neg_items):
    n_users, d = user_emb.shape
    n_items = item_emb.shape[0]
    n = n_users + n_items
    n_pad = ((n + 1023) // 1024) * 1024
    e = adj_values.shape[0]
    e_pad = ((e + 16383) // 16384) * 16384
    crows = e_pad // 128
    chunks_per_tile = crows // NS
    bsz = users.shape[0]

    row = adj_indices[0].astype(jnp.int32)
    col = adj_indices[1].astype(jnp.int32)
    npad_rows = n_pad - n
    ar = jnp.arange(e_pad - e, dtype=jnp.int32)
    spread = n + (ar % npad_rows)
    row_p = jnp.concatenate([row, spread])
    col_p = jnp.concatenate([col, spread])
    val_p = jnp.concatenate([adj_values, jnp.zeros((e_pad - e,), jnp.float32)])
    epack = jnp.stack(
        [col_p.reshape(crows, 128), row_p.reshape(crows, 128),
         lax.bitcast_convert_type(val_p, jnp.int32).reshape(crows, 128)],
        axis=1)
    zeros = jnp.zeros((n_pad, HALF), jnp.float32)

    all_emb = jnp.concatenate([user_emb, item_emb], axis=0)
    all_pad = jnp.pad(all_emb, ((0, npad_rows), (0, 0)))
    x0p = all_pad[:, :HALF].reshape(n_pad // 4, 128)
    x1p = all_pad[:, HALF:].reshape(n_pad // 4, 128)

    eye4 = jnp.eye(4, dtype=jnp.float32)
    bdone = jnp.kron(eye4, jnp.ones((HALF, HALF), jnp.float32))
    bdone2 = jnp.kron(jnp.eye(2, dtype=jnp.float32),
                      jnp.ones((64, 64), jnp.float32))

    spmm = _make_spmm(n_pad, chunks_per_tile)
    halves = [(x0p, x1p)]
    for k in range(W1.shape[0]):
        xa, xb = halves[-1]
        sp = spmm(xa.reshape(n_pad, HALF), xb.reshape(n_pad, HALF),
                  epack, zeros)
        blocks = []
        for c in range(2):
            for w in (W1[k], W2[k]):
                blocks.append(jnp.kron(
                    eye4, w[HALF * c:HALF * (c + 1), :HALF].T))
                blocks.append(jnp.kron(
                    eye4, w[HALF * c:HALF * (c + 1), HALF:].T))
        wstack = jnp.stack(blocks)
        bsum = b1[k] + b2[k]
        bias = jnp.stack([jnp.tile(bsum[:HALF], 4), jnp.tile(bsum[HALF:], 4)])
        bias = bias.reshape(2, 1, 128)
        o0, o1 = _dense_layer(sp.reshape(2 * n_pad // 4, 128), xa, xb,
                              wstack, bdone, bias, n_pad)
        halves.append((o0, o1))

    idx_u = users.astype(jnp.int32)
    idx_p = pos_items.astype(jnp.int32) + n_users
    idx_n = neg_items.astype(jnp.int32) + n_users
    idx = jnp.stack([idx_u, idx_p, idx_n]).reshape(3, bsz // 128, 128)

    tables = []
    for (a, b) in halves:
        tables.append(a.reshape(n_pad, HALF))
        tables.append(b.reshape(n_pad, HALF))
    g = _make_gather(n_pad, bsz)(*tables, idx)
    loss_a, bpr_a = _loss_tc(g.reshape(4, 3, bsz // 2, 128), bsz, bdone2)
    return (loss_a[0, 0], bpr_a[0, 0])
